# Initial kernel scaffold; baseline (speedup 1.0000x reference)
import jax
import jax.numpy as jnp
from jax.experimental import pallas as pl


def kernel(h, cluster_ids, W1, b1, W2, b2):
    raise NotImplementedError("write your pallas kernel here")


# speedup vs baseline: 1.7374x; 1.7374x over previous
import functools

import jax
import jax.numpy as jnp
from jax import lax
from jax.experimental import pallas as pl
from jax.experimental.pallas import tpu as pltpu

NUM_CLUSTERS = 8
HIDDEN_DIM = 2048
BOTTLENECK_DIM = 512
N_TOKENS = 4096
ROW_BLOCK = 512


def _adapter_body(h_bf_ref, h_f32_ref, cid_ref, w1_ref, b1_ref, w2_ref,
                  b2_ref, out_ref):
    e = pl.program_id(1)

    @pl.when(e == 0)
    def _init():
        out_ref[...] = jnp.zeros_like(out_ref)

    hb = h_bf_ref[...]
    w1 = w1_ref[0]
    z = lax.dot_general(hb, w1, (((1,), (1,)), ((), ())),
                        preferred_element_type=jnp.float32)
    z = z + b1_ref[0]
    z = 0.5 * z * (1.0 + lax.erf(z * 0.7071067811865476))
    zb = z.astype(jnp.bfloat16)
    w2 = w2_ref[0]
    delta = lax.dot_general(zb, w2, (((1,), (1,)), ((), ())),
                            preferred_element_type=jnp.float32)
    delta = delta + b2_ref[0]
    mask = cid_ref[...] == e
    out_ref[...] = jnp.where(mask, h_f32_ref[...] + delta, out_ref[...])


def kernel(h, cluster_ids, W1, b1, W2, b2):
    h_bf = h.astype(jnp.bfloat16)
    W1_bf = W1.astype(jnp.bfloat16)
    W2_bf = W2.astype(jnp.bfloat16)
    cid2 = cluster_ids.astype(jnp.int32).reshape(N_TOKENS, 1)
    b1r = b1.reshape(NUM_CLUSTERS, 1, BOTTLENECK_DIM)
    b2r = b2.reshape(NUM_CLUSTERS, 1, HIDDEN_DIM)

    n_tiles = N_TOKENS // ROW_BLOCK
    grid = (n_tiles, NUM_CLUSTERS)

    out = pl.pallas_call(
        _adapter_body,
        grid=grid,
        in_specs=[
            pl.BlockSpec((ROW_BLOCK, HIDDEN_DIM), lambda i, e: (i, 0)),
            pl.BlockSpec((ROW_BLOCK, HIDDEN_DIM), lambda i, e: (i, 0)),
            pl.BlockSpec((ROW_BLOCK, 1), lambda i, e: (i, 0)),
            pl.BlockSpec((1, BOTTLENECK_DIM, HIDDEN_DIM), lambda i, e: (e, 0, 0)),
            pl.BlockSpec((1, 1, BOTTLENECK_DIM), lambda i, e: (e, 0, 0)),
            pl.BlockSpec((1, HIDDEN_DIM, BOTTLENECK_DIM), lambda i, e: (e, 0, 0)),
            pl.BlockSpec((1, 1, HIDDEN_DIM), lambda i, e: (e, 0, 0)),
        ],
        out_specs=pl.BlockSpec((ROW_BLOCK, HIDDEN_DIM), lambda i, e: (i, 0)),
        out_shape=jax.ShapeDtypeStruct((N_TOKENS, HIDDEN_DIM), jnp.float32),
        compiler_params=pltpu.CompilerParams(
            dimension_semantics=("parallel", "arbitrary")),
    )(h_bf, h, cid2, W1_bf, b1r, W2_bf, b2r)
    return out


# speedup vs baseline: 1.8647x; 1.0733x over previous
import functools

import jax
import jax.numpy as jnp
from jax import lax
from jax.experimental import pallas as pl
from jax.experimental.pallas import tpu as pltpu

NUM_CLUSTERS = 8
HIDDEN_DIM = 2048
BOTTLENECK_DIM = 512
N_TOKENS = 4096
ROW_BLOCK = 512
M_TILES = N_TOKENS // ROW_BLOCK
GRID_T = M_TILES + NUM_CLUSTERS - 1


def _route_metadata(cid):
    counts = jnp.bincount(cid, length=NUM_CLUSTERS).astype(jnp.int32)
    off = jnp.concatenate(
        [jnp.zeros((1,), jnp.int32), jnp.cumsum(counts).astype(jnp.int32)])
    sort_idx = jnp.argsort(cid).astype(jnp.int32)
    inv_perm = jnp.zeros((N_TOKENS,), jnp.int32).at[sort_idx].set(
        jnp.arange(N_TOKENS, dtype=jnp.int32))
    start_tile = off[:-1] // ROW_BLOCK
    end_tile = jnp.where(counts > 0,
                         (off[1:] + ROW_BLOCK - 1) // ROW_BLOCK, start_tile)
    ntiles = end_tile - start_tile
    cum_t = jnp.cumsum(ntiles)
    t = jnp.arange(GRID_T, dtype=jnp.int32)
    g = jnp.searchsorted(cum_t, t, side="right").astype(jnp.int32)
    g = jnp.minimum(g, NUM_CLUSTERS - 1)
    prev = jnp.where(g > 0, cum_t[jnp.maximum(g - 1, 0)], 0).astype(jnp.int32)
    tile = jnp.clip(start_tile[g] + (t - prev), 0, M_TILES - 1)
    return sort_idx, inv_perm, off, g, tile


def _gmm_body(g_ref, tile_ref, off_ref, hs_bf_ref, hs_f32_ref, w1_ref,
              b1_ref, w2_ref, b2_ref, out_ref):
    t = pl.program_id(0)
    g = g_ref[t]
    tile = tile_ref[t]
    first = jnp.logical_or(t == 0, tile != tile_ref[jnp.maximum(t - 1, 0)])

    @pl.when(first)
    def _init():
        out_ref[...] = jnp.zeros_like(out_ref)

    z = lax.dot_general(hs_bf_ref[...], w1_ref[0], (((1,), (1,)), ((), ())),
                        preferred_element_type=jnp.float32)
    z = z + b1_ref[0]
    z = 0.5 * z * (1.0 + lax.erf(z * 0.7071067811865476))
    delta = lax.dot_general(z.astype(jnp.bfloat16), w2_ref[0],
                            (((1,), (1,)), ((), ())),
                            preferred_element_type=jnp.float32)
    delta = delta + b2_ref[0]
    row = lax.broadcasted_iota(jnp.int32, (ROW_BLOCK, 1), 0) + tile * ROW_BLOCK
    mask = jnp.logical_and(row >= off_ref[g], row < off_ref[g + 1])
    out_ref[...] = jnp.where(mask, hs_f32_ref[...] + delta, out_ref[...])


def _grouped_adapter(hs_bf, hs_f32, W1_bf, b1r, W2_bf, b2r, off, g, tile):
    grid_spec = pltpu.PrefetchScalarGridSpec(
        num_scalar_prefetch=3,
        grid=(GRID_T,),
        in_specs=[
            pl.BlockSpec((ROW_BLOCK, HIDDEN_DIM),
                         lambda t, gr, tr, orf: (tr[t], 0)),
            pl.BlockSpec((ROW_BLOCK, HIDDEN_DIM),
                         lambda t, gr, tr, orf: (tr[t], 0)),
            pl.BlockSpec((1, BOTTLENECK_DIM, HIDDEN_DIM),
                         lambda t, gr, tr, orf: (gr[t], 0, 0)),
            pl.BlockSpec((1, 1, BOTTLENECK_DIM),
                         lambda t, gr, tr, orf: (gr[t], 0, 0)),
            pl.BlockSpec((1, HIDDEN_DIM, BOTTLENECK_DIM),
                         lambda t, gr, tr, orf: (gr[t], 0, 0)),
            pl.BlockSpec((1, 1, HIDDEN_DIM),
                         lambda t, gr, tr, orf: (gr[t], 0, 0)),
        ],
        out_specs=pl.BlockSpec((ROW_BLOCK, HIDDEN_DIM),
                               lambda t, gr, tr, orf: (tr[t], 0)),
    )
    return pl.pallas_call(
        _gmm_body,
        grid_spec=grid_spec,
        out_shape=jax.ShapeDtypeStruct((N_TOKENS, HIDDEN_DIM), jnp.float32),
        compiler_params=pltpu.CompilerParams(
            dimension_semantics=("arbitrary",)),
    )(g, tile, off, hs_bf, hs_f32, W1_bf, b1r, W2_bf, b2r)


def kernel(h, cluster_ids, W1, b1, W2, b2):
    cid = cluster_ids.astype(jnp.int32)
    sort_idx, inv_perm, off, g, tile = _route_metadata(cid)

    hs_f32 = jnp.take(h, sort_idx, axis=0)
    hs_bf = hs_f32.astype(jnp.bfloat16)

    W1_bf = W1.astype(jnp.bfloat16)
    W2_bf = W2.astype(jnp.bfloat16)
    b1r = b1.reshape(NUM_CLUSTERS, 1, BOTTLENECK_DIM)
    b2r = b2.reshape(NUM_CLUSTERS, 1, HIDDEN_DIM)

    out_sorted = _grouped_adapter(hs_bf, hs_f32, W1_bf, b1r, W2_bf, b2r,
                                  off, g, tile)
    return jnp.take(out_sorted, inv_perm, axis=0)


# speedup vs baseline: 2.1649x; 1.1610x over previous
import functools

import jax
import jax.numpy as jnp
from jax import lax
from jax.experimental import pallas as pl
from jax.experimental.pallas import tpu as pltpu

NUM_CLUSTERS = 8
HIDDEN_DIM = 2048
BOTTLENECK_DIM = 512
N_TOKENS = 4096
ROW_BLOCK = 512
M_TILES = N_TOKENS // ROW_BLOCK
GRID_T = M_TILES + NUM_CLUSTERS - 1


def _route_metadata(cid):
    counts = jnp.bincount(cid, length=NUM_CLUSTERS).astype(jnp.int32)
    off = jnp.concatenate(
        [jnp.zeros((1,), jnp.int32), jnp.cumsum(counts).astype(jnp.int32)])
    sort_idx = jnp.argsort(cid).astype(jnp.int32)
    inv_perm = jnp.zeros((N_TOKENS,), jnp.int32).at[sort_idx].set(
        jnp.arange(N_TOKENS, dtype=jnp.int32))
    start_tile = off[:-1] // ROW_BLOCK
    end_tile = jnp.where(counts > 0,
                         (off[1:] + ROW_BLOCK - 1) // ROW_BLOCK, start_tile)
    ntiles = end_tile - start_tile
    cum_t = jnp.cumsum(ntiles)
    t = jnp.arange(GRID_T, dtype=jnp.int32)
    g = jnp.searchsorted(cum_t, t, side="right").astype(jnp.int32)
    g = jnp.minimum(g, NUM_CLUSTERS - 1)
    prev = jnp.where(g > 0, cum_t[jnp.maximum(g - 1, 0)], 0).astype(jnp.int32)
    tile = jnp.clip(start_tile[g] + (t - prev), 0, M_TILES - 1)
    return sort_idx, inv_perm, off, g, tile


def _gmm_body(g_ref, tile_ref, off_ref, hs_ref, w1_ref, b1_ref, w2_ref,
              b2_ref, out_ref):
    t = pl.program_id(0)
    g = g_ref[t]
    tile = tile_ref[t]
    first = jnp.logical_or(t == 0, tile != tile_ref[jnp.maximum(t - 1, 0)])

    @pl.when(first)
    def _init():
        out_ref[...] = jnp.zeros_like(out_ref)

    z = lax.dot_general(hs_ref[...], w1_ref[0], (((1,), (1,)), ((), ())),
                        precision=lax.Precision.DEFAULT,
                        preferred_element_type=jnp.float32)
    z = z + b1_ref[0]
    z = 0.5 * z * (1.0 + lax.erf(z * 0.7071067811865476))
    delta = lax.dot_general(z, w2_ref[0], (((1,), (1,)), ((), ())),
                            precision=lax.Precision.DEFAULT,
                            preferred_element_type=jnp.float32)
    delta = delta + b2_ref[0]
    row = lax.broadcasted_iota(jnp.int32, (ROW_BLOCK, 1), 0) + tile * ROW_BLOCK
    mask = jnp.logical_and(row >= off_ref[g], row < off_ref[g + 1])
    out_ref[...] = jnp.where(mask, delta, out_ref[...])


def _grouped_delta(hs, W1, b1r, W2, b2r, off, g, tile):
    grid_spec = pltpu.PrefetchScalarGridSpec(
        num_scalar_prefetch=3,
        grid=(GRID_T,),
        in_specs=[
            pl.BlockSpec((ROW_BLOCK, HIDDEN_DIM),
                         lambda t, gr, tr, orf: (tr[t], 0)),
            pl.BlockSpec((1, BOTTLENECK_DIM, HIDDEN_DIM),
                         lambda t, gr, tr, orf: (gr[t], 0, 0)),
            pl.BlockSpec((1, 1, BOTTLENECK_DIM),
                         lambda t, gr, tr, orf: (gr[t], 0, 0)),
            pl.BlockSpec((1, HIDDEN_DIM, BOTTLENECK_DIM),
                         lambda t, gr, tr, orf: (gr[t], 0, 0)),
            pl.BlockSpec((1, 1, HIDDEN_DIM),
                         lambda t, gr, tr, orf: (gr[t], 0, 0)),
        ],
        out_specs=pl.BlockSpec((ROW_BLOCK, HIDDEN_DIM),
                               lambda t, gr, tr, orf: (tr[t], 0)),
    )
    return pl.pallas_call(
        _gmm_body,
        grid_spec=grid_spec,
        out_shape=jax.ShapeDtypeStruct((N_TOKENS, HIDDEN_DIM), jnp.float32),
        compiler_params=pltpu.CompilerParams(
            dimension_semantics=("arbitrary",)),
    )(g, tile, off, hs, W1, b1r, W2, b2r)


def kernel(h, cluster_ids, W1, b1, W2, b2):
    cid = cluster_ids.astype(jnp.int32)
    sort_idx, inv_perm, off, g, tile = _route_metadata(cid)

    hs = jnp.take(h, sort_idx, axis=0)
    b1r = b1.reshape(NUM_CLUSTERS, 1, BOTTLENECK_DIM)
    b2r = b2.reshape(NUM_CLUSTERS, 1, HIDDEN_DIM)

    delta_sorted = _grouped_delta(hs, W1, b1r, W2, b2r, off, g, tile)
    return h + jnp.take(delta_sorted, inv_perm, axis=0)


# speedup vs baseline: 2.2031x; 1.0176x over previous
import functools

import jax
import jax.numpy as jnp
from jax import lax
from jax.experimental import pallas as pl
from jax.experimental.pallas import tpu as pltpu

NUM_CLUSTERS = 8
HIDDEN_DIM = 2048
BOTTLENECK_DIM = 512
N_TOKENS = 4096
ROW_BLOCK = 512
M_TILES = N_TOKENS // ROW_BLOCK
GRID_T = M_TILES + NUM_CLUSTERS - 1


def _route_metadata(cid):
    onehot = (cid[:, None] == jnp.arange(NUM_CLUSTERS)[None, :]).astype(
        jnp.int32)
    cum = jnp.cumsum(onehot, axis=0)
    counts = cum[-1]
    off = jnp.concatenate(
        [jnp.zeros((1,), jnp.int32), jnp.cumsum(counts).astype(jnp.int32)])
    rank = off[cid] + jnp.take_along_axis(cum, cid[:, None], axis=1)[:, 0] - 1
    start_tile = off[:-1] // ROW_BLOCK
    end_tile = jnp.where(counts > 0,
                         (off[1:] + ROW_BLOCK - 1) // ROW_BLOCK, start_tile)
    ntiles = end_tile - start_tile
    cum_t = jnp.cumsum(ntiles)
    t = jnp.arange(GRID_T, dtype=jnp.int32)
    g = jnp.searchsorted(cum_t, t, side="right").astype(jnp.int32)
    g = jnp.minimum(g, NUM_CLUSTERS - 1)
    prev = jnp.where(g > 0, cum_t[jnp.maximum(g - 1, 0)], 0).astype(jnp.int32)
    tile = jnp.clip(start_tile[g] + (t - prev), 0, M_TILES - 1)
    return rank, off, g, tile


def _gmm_body(g_ref, tile_ref, off_ref, hs_ref, w1_ref, b1_ref, w2_ref,
              b2_ref, out_ref):
    t = pl.program_id(0)
    g = g_ref[t]
    tile = tile_ref[t]
    first = jnp.logical_or(t == 0, tile != tile_ref[jnp.maximum(t - 1, 0)])

    @pl.when(first)
    def _init():
        out_ref[...] = jnp.zeros_like(out_ref)

    z = lax.dot_general(hs_ref[...], w1_ref[0], (((1,), (1,)), ((), ())),
                        precision=lax.Precision.DEFAULT,
                        preferred_element_type=jnp.float32)
    z = z + b1_ref[0]
    z = 0.5 * z * (1.0 + lax.erf(z * 0.7071067811865476))
    delta = lax.dot_general(z, w2_ref[0], (((1,), (1,)), ((), ())),
                            precision=lax.Precision.DEFAULT,
                            preferred_element_type=jnp.float32)
    delta = delta + b2_ref[0]
    row = lax.broadcasted_iota(jnp.int32, (ROW_BLOCK, 1), 0) + tile * ROW_BLOCK
    mask = jnp.logical_and(row >= off_ref[g], row < off_ref[g + 1])
    out_ref[...] = jnp.where(mask, delta, out_ref[...])


def _grouped_delta(hs, W1, b1r, W2, b2r, off, g, tile):
    grid_spec = pltpu.PrefetchScalarGridSpec(
        num_scalar_prefetch=3,
        grid=(GRID_T,),
        in_specs=[
            pl.BlockSpec((ROW_BLOCK, HIDDEN_DIM),
                         lambda t, gr, tr, orf: (tr[t], 0)),
            pl.BlockSpec((1, BOTTLENECK_DIM, HIDDEN_DIM),
                         lambda t, gr, tr, orf: (gr[t], 0, 0)),
            pl.BlockSpec((1, 1, BOTTLENECK_DIM),
                         lambda t, gr, tr, orf: (gr[t], 0, 0)),
            pl.BlockSpec((1, HIDDEN_DIM, BOTTLENECK_DIM),
                         lambda t, gr, tr, orf: (gr[t], 0, 0)),
            pl.BlockSpec((1, 1, HIDDEN_DIM),
                         lambda t, gr, tr, orf: (gr[t], 0, 0)),
        ],
        out_specs=pl.BlockSpec((ROW_BLOCK, HIDDEN_DIM),
                               lambda t, gr, tr, orf: (tr[t], 0)),
    )
    return pl.pallas_call(
        _gmm_body,
        grid_spec=grid_spec,
        out_shape=jax.ShapeDtypeStruct((N_TOKENS, HIDDEN_DIM), jnp.float32),
        compiler_params=pltpu.CompilerParams(
            dimension_semantics=("arbitrary",)),
    )(g, tile, off, hs, W1, b1r, W2, b2r)


def kernel(h, cluster_ids, W1, b1, W2, b2):
    cid = cluster_ids.astype(jnp.int32)
    rank, off, g, tile = _route_metadata(cid)

    hs = jnp.zeros_like(h).at[rank].set(h)
    b1r = b1.reshape(NUM_CLUSTERS, 1, BOTTLENECK_DIM)
    b2r = b2.reshape(NUM_CLUSTERS, 1, HIDDEN_DIM)

    delta_sorted = _grouped_delta(hs, W1, b1r, W2, b2r, off, g, tile)
    return h + jnp.take(delta_sorted, rank, axis=0)
